# baseline (speedup 1.0000x reference)
import functools

import jax
import jax.numpy as jnp
from jax import lax
from jax.experimental import pallas as pl
from jax.experimental.pallas import tpu as pltpu
from jax.experimental.pallas import tpu_sc as plsc

NUM_SHAPES = 100000
LATENT_DIM = 256
BATCH = 16384

_NC = 2
_NS = 16
_NW = _NC * _NS
_BPW = BATCH // _NW
_CHUNK = 32
_NCHUNK = _BPW // _CHUNK
_NBUF = 12


def _make_gather():
    mesh = plsc.VectorSubcoreMesh(core_axis_name="c", subcore_axis_name="s")

    @functools.partial(
        pl.kernel,
        mesh=mesh,
        out_type=jax.ShapeDtypeStruct((BATCH, LATENT_DIM), jnp.float32),
        scratch_types=[
            pltpu.VMEM((_BPW,), jnp.int32),
            pltpu.VMEM((_NBUF, _CHUNK, LATENT_DIM), jnp.float32),
        ]
        + [pltpu.SemaphoreType.DMA] * (1 + 2 * _NBUF),
    )
    def gather_kernel(idx_hbm, table_hbm, out_hbm, idx_v, rows_v, *sems):
        isem = sems[0]
        gsem = sems[1:1 + _NBUF]
        osem = sems[1 + _NBUF:]
        wid = lax.axis_index("s") * _NC + lax.axis_index("c")
        base = wid * _BPW
        idx_cp = pltpu.async_copy(idx_hbm.at[pl.ds(base, _BPW)], idx_v, isem)
        gathers = [None] * _NCHUNK
        outs = [None] * _NCHUNK

        def fire_gather(c):
            if c == 0:
                idx_cp.wait()
            gathers[c] = pltpu.async_copy(
                table_hbm.at[idx_v.at[pl.ds(c * _CHUNK, _CHUNK)]],
                rows_v.at[c % _NBUF],
                gsem[c % _NBUF],
            )

        for c in range(min(_NBUF, _NCHUNK)):
            fire_gather(c)
        for c in range(_NCHUNK):
            gathers[c].wait()
            outs[c] = pltpu.async_copy(
                rows_v.at[c % _NBUF],
                out_hbm.at[pl.ds(base + c * _CHUNK, _CHUNK)],
                osem[c % _NBUF],
            )
            nxt = c + _NBUF
            if nxt < _NCHUNK:
                outs[c].wait()
                fire_gather(nxt)
        for c in range(max(0, _NCHUNK - _NBUF), _NCHUNK):
            outs[c].wait()

    return gather_kernel


_gather = _make_gather()


@jax.jit
def kernel(indices, weight):
    return _gather(indices, weight)


# speedup vs baseline: 1.0135x; 1.0135x over previous
import functools

import jax
import jax.numpy as jnp
from jax import lax
from jax.experimental import pallas as pl
from jax.experimental.pallas import tpu as pltpu
from jax.experimental.pallas import tpu_sc as plsc

NUM_SHAPES = 100000
LATENT_DIM = 256
BATCH = 16384

_NC = 2
_NS = 16
_NW = _NC * _NS
_BPW = BATCH // _NW
_CHUNK = 64
_NCHUNK = _BPW // _CHUNK
_NBUF = 6


def _make_gather():
    mesh = plsc.VectorSubcoreMesh(core_axis_name="c", subcore_axis_name="s")

    @functools.partial(
        pl.kernel,
        mesh=mesh,
        out_type=jax.ShapeDtypeStruct((BATCH, LATENT_DIM), jnp.float32),
        scratch_types=[
            pltpu.VMEM((_BPW,), jnp.int32),
            pltpu.VMEM((_NBUF, _CHUNK, LATENT_DIM), jnp.float32),
        ]
        + [pltpu.SemaphoreType.DMA] * (1 + 2 * _NBUF),
    )
    def gather_kernel(idx_hbm, table_hbm, out_hbm, idx_v, rows_v, *sems):
        isem = sems[0]
        gsem = sems[1:1 + _NBUF]
        osem = sems[1 + _NBUF:]
        wid = lax.axis_index("s") * _NC + lax.axis_index("c")
        base = wid * _BPW
        idx_cp = pltpu.async_copy(idx_hbm.at[pl.ds(base, _BPW)], idx_v, isem)
        gathers = [None] * _NCHUNK
        outs = [None] * _NCHUNK

        def fire_gather(c):
            if c == 0:
                idx_cp.wait()
            gathers[c] = pltpu.async_copy(
                table_hbm.at[idx_v.at[pl.ds(c * _CHUNK, _CHUNK)]],
                rows_v.at[c % _NBUF],
                gsem[c % _NBUF],
            )

        for c in range(min(_NBUF, _NCHUNK)):
            fire_gather(c)
        for c in range(_NCHUNK):
            gathers[c].wait()
            outs[c] = pltpu.async_copy(
                rows_v.at[c % _NBUF],
                out_hbm.at[pl.ds(base + c * _CHUNK, _CHUNK)],
                osem[c % _NBUF],
            )
            nxt = c + _NBUF
            if nxt < _NCHUNK:
                outs[c].wait()
                fire_gather(nxt)
        for c in range(max(0, _NCHUNK - _NBUF), _NCHUNK):
            outs[c].wait()

    return gather_kernel


_gather = _make_gather()


@jax.jit
def kernel(indices, weight):
    return _gather(indices, weight)


# speedup vs baseline: 1.0173x; 1.0038x over previous
import functools

import jax
import jax.numpy as jnp
from jax import lax
from jax.experimental import pallas as pl
from jax.experimental.pallas import tpu as pltpu
from jax.experimental.pallas import tpu_sc as plsc

NUM_SHAPES = 100000
LATENT_DIM = 256
BATCH = 16384

_NC = 2
_NS = 16
_NW = _NC * _NS
_BPW = BATCH // _NW
_CHUNK = 64
_NCHUNK = _BPW // _CHUNK
_NBUF = 6


def _make_gather():
    mesh = plsc.VectorSubcoreMesh(core_axis_name="c", subcore_axis_name="s")

    @functools.partial(
        pl.kernel,
        mesh=mesh,
        out_type=jax.ShapeDtypeStruct((BATCH, LATENT_DIM), jnp.float32),
        scratch_types=[
            pltpu.VMEM((_BPW,), jnp.int32),
            pltpu.VMEM((_NBUF, _CHUNK, LATENT_DIM), jnp.float32),
        ]
        + [pltpu.SemaphoreType.DMA] * (1 + 2 * _NBUF),
    )
    def gather_kernel(idx_hbm, table_hbm, out_hbm, idx_v, rows_v, *sems):
        isem = sems[0]
        gsem = sems[1:1 + _NBUF]
        osem = sems[1 + _NBUF:]
        wid = lax.axis_index("c") * _NS + lax.axis_index("s")
        base = wid * _BPW
        idx_cp = pltpu.async_copy(idx_hbm.at[pl.ds(base, _BPW)], idx_v, isem)
        gathers = [None] * _NCHUNK
        outs = [None] * _NCHUNK

        def fire_gather(c):
            if c == 0:
                idx_cp.wait()
            gathers[c] = pltpu.async_copy(
                table_hbm.at[idx_v.at[pl.ds(c * _CHUNK, _CHUNK)]],
                rows_v.at[c % _NBUF],
                gsem[c % _NBUF],
            )

        for c in range(min(_NBUF, _NCHUNK)):
            fire_gather(c)
        for c in range(_NCHUNK):
            gathers[c].wait()
            outs[c] = pltpu.async_copy(
                rows_v.at[c % _NBUF],
                out_hbm.at[pl.ds(base + c * _CHUNK, _CHUNK)],
                osem[c % _NBUF],
            )
            nxt = c + _NBUF
            if nxt < _NCHUNK:
                outs[c].wait()
                fire_gather(nxt)
        for c in range(max(0, _NCHUNK - _NBUF), _NCHUNK):
            outs[c].wait()

    return gather_kernel


_gather = _make_gather()


@jax.jit
def kernel(indices, weight):
    return _gather(indices, weight)
